# baseline (speedup 1.0000x reference)
import jax
import jax.numpy as jnp
from jax.experimental import pallas as pl
from jax.experimental.pallas import tpu as pltpu

NF = 100000
E = 32
K = 64
BT = 32
CH = 2048
NCH = 49
FPAD = CH * NCH
MASK = 2**31 - 1
SIGN = -(2**31)


def _body(emb_ref, w_ref, b_ref, tab_ref, out_ref, keys_ref):
    emb = emb_ref[...]

    def phase_a(c, carry):
        wc = w_ref[:, pl.ds(c * CH, CH)]
        logits = jax.lax.dot_general(
            emb, wc, (((1,), (0,)), ((), ())),
            preferred_element_type=jnp.float32)
        logits = logits + b_ref[:, pl.ds(c * CH, CH)]
        i = jax.lax.bitcast_convert_type(logits, jnp.int32)
        keys_ref[:, pl.ds(c * CH, CH)] = jnp.where(i < 0, i ^ MASK, i)
        return carry

    jax.lax.fori_loop(0, NCH, phase_a, 0)

    def bit_step(t, vt):
        bit = 31 - t
        cand_u = vt | (jnp.int32(1) << bit)
        candk = cand_u ^ SIGN

        def count_chunk(c, cnt):
            k = keys_ref[:, pl.ds(c * CH, CH)]
            return cnt + jnp.sum((k >= candk).astype(jnp.int32), axis=1,
                                 keepdims=True)

        cnt = jax.lax.fori_loop(0, NCH, count_chunk,
                                jnp.zeros((BT, 1), jnp.int32))
        return jnp.where(cnt >= K, cand_u, vt)

    vt = jax.lax.fori_loop(0, 32, bit_step, jnp.zeros((BT, 1), jnp.int32))
    tk = vt ^ SIGN

    def phase_c(c, acc):
        k = keys_ref[:, pl.ds(c * CH, CH)]
        i = jnp.where(k < 0, k ^ MASK, k)
        logit = jax.lax.bitcast_convert_type(i, jnp.float32)
        w = jnp.where(k >= tk, 1.0 / (1.0 + jnp.exp(-logit)), 0.0)
        tc = tab_ref[:, pl.ds(c * CH, CH)]
        return acc + jax.lax.dot_general(
            w, tc, (((1,), (1,)), ((), ())),
            preferred_element_type=jnp.float32)

    out_ref[...] = jax.lax.fori_loop(0, NCH, phase_c,
                                     jnp.zeros((BT, E), jnp.float32))


def kernel(embed, W_enc, b_enc, table):
    B = embed.shape[0]
    pad = FPAD - NF
    wp = jnp.pad(W_enc.T, ((0, 0), (0, pad)))
    bp = jnp.pad(b_enc, (0, pad), constant_values=-1e30).reshape(1, FPAD)
    tp = jnp.pad(table.T, ((0, 0), (0, pad)))
    return pl.pallas_call(
        _body,
        grid=(B // BT,),
        in_specs=[
            pl.BlockSpec((BT, E), lambda i: (i, 0)),
            pl.BlockSpec((E, FPAD), lambda i: (0, 0)),
            pl.BlockSpec((1, FPAD), lambda i: (0, 0)),
            pl.BlockSpec((E, FPAD), lambda i: (0, 0)),
        ],
        out_specs=pl.BlockSpec((BT, E), lambda i: (i, 0)),
        out_shape=jax.ShapeDtypeStruct((B, E), jnp.float32),
        scratch_shapes=[pltpu.VMEM((BT, FPAD), jnp.int32)],
        compiler_params=pltpu.CompilerParams(
            dimension_semantics=("parallel",)),
    )(embed, wp, bp, tp)


# speedup vs baseline: 1.1896x; 1.1896x over previous
import jax
import jax.numpy as jnp
from jax.experimental import pallas as pl
from jax.experimental.pallas import tpu as pltpu

NF = 100000
E = 32
K = 64
BT = 32
CH = 2048
NCH = 49
FPAD = CH * NCH
MASK = 2**31 - 1
SIGN = -(2**31)


def _body(emb_ref, w_ref, b_ref, tab_ref, out_ref, keys_ref):
    emb = emb_ref[...]

    def phase_a(c, carry):
        hi, lo = carry
        wc = w_ref[:, pl.ds(c * CH, CH)]
        logits = jax.lax.dot_general(
            emb, wc, (((1,), (0,)), ((), ())),
            preferred_element_type=jnp.float32)
        logits = logits + b_ref[:, pl.ds(c * CH, CH)]
        i = jax.lax.bitcast_convert_type(logits, jnp.int32)
        k = jnp.where(i < 0, i ^ MASK, i)
        keys_ref[:, pl.ds(c * CH, CH)] = k
        m1 = jnp.max(k, axis=1, keepdims=True)
        m2 = jnp.max(jnp.where(k == m1, SIGN, k), axis=1, keepdims=True)
        return jnp.maximum(hi, m1), jnp.minimum(lo, m2)

    hi, lo = jax.lax.fori_loop(
        0, NCH, phase_a,
        (jnp.full((BT, 1), SIGN, jnp.int32),
         jnp.full((BT, 1), MASK, jnp.int32)))

    def count_ge(cand):
        def count_chunk(c, cnt):
            k = keys_ref[:, pl.ds(c * CH, CH)]
            return cnt + jnp.sum((k >= cand).astype(jnp.int32), axis=1,
                                 keepdims=True)
        return jax.lax.fori_loop(0, NCH, count_chunk,
                                 jnp.zeros((BT, 1), jnp.int32))

    def bs_cond(carry):
        hi, lo = carry
        return jnp.any(lo < hi)

    def bs_step(carry):
        hi, lo = carry
        half = jax.lax.shift_right_logical(hi - lo, 1)
        mid = ((hi ^ SIGN) - half) ^ SIGN
        cnt = count_ge(mid)
        ok = cnt >= K
        lo = jnp.where(ok, mid, lo)
        hi = jnp.where(ok, hi, ((mid ^ SIGN) - 1) ^ SIGN)
        return hi, lo

    hi, lo = jax.lax.while_loop(bs_cond, bs_step, (hi, lo))
    tk = lo

    def phase_c(c, acc):
        k = keys_ref[:, pl.ds(c * CH, CH)]
        i = jnp.where(k < 0, k ^ MASK, k)
        logit = jax.lax.bitcast_convert_type(i, jnp.float32)
        w = jnp.where(k >= tk, 1.0 / (1.0 + jnp.exp(-logit)), 0.0)
        tc = tab_ref[:, pl.ds(c * CH, CH)]
        return acc + jax.lax.dot_general(
            w, tc, (((1,), (1,)), ((), ())),
            preferred_element_type=jnp.float32)

    out_ref[...] = jax.lax.fori_loop(0, NCH, phase_c,
                                     jnp.zeros((BT, E), jnp.float32))


def kernel(embed, W_enc, b_enc, table):
    B = embed.shape[0]
    pad = FPAD - NF
    wp = jnp.pad(W_enc.T, ((0, 0), (0, pad)))
    bp = jnp.pad(b_enc, (0, pad), constant_values=-1e30).reshape(1, FPAD)
    tp = jnp.pad(table.T, ((0, 0), (0, pad)))
    return pl.pallas_call(
        _body,
        grid=(B // BT,),
        in_specs=[
            pl.BlockSpec((BT, E), lambda i: (i, 0)),
            pl.BlockSpec((E, FPAD), lambda i: (0, 0)),
            pl.BlockSpec((1, FPAD), lambda i: (0, 0)),
            pl.BlockSpec((E, FPAD), lambda i: (0, 0)),
        ],
        out_specs=pl.BlockSpec((BT, E), lambda i: (i, 0)),
        out_shape=jax.ShapeDtypeStruct((B, E), jnp.float32),
        scratch_shapes=[pltpu.VMEM((BT, FPAD), jnp.int32)],
        compiler_params=pltpu.CompilerParams(
            dimension_semantics=("parallel",)),
    )(embed, wp, bp, tp)


# speedup vs baseline: 2.0508x; 1.7240x over previous
import jax
import jax.numpy as jnp
from jax.experimental import pallas as pl
from jax.experimental.pallas import tpu as pltpu

NF = 100000
E = 32
K = 64
BT = 32
CH = 2048
NCH = 49
FPAD = CH * NCH
CH2 = 7168
NCH2 = 14
MASK = 2**31 - 1
SIGN = -(2**31)


def _body(emb_ref, w_ref, b_ref, tab_ref, out_ref, keys_ref):
    emb = emb_ref[...]

    def phase_a(c, carry):
        hi, lo = carry
        wc = w_ref[:, pl.ds(c * CH, CH)]
        logits = jax.lax.dot_general(
            emb, wc, (((1,), (0,)), ((), ())),
            preferred_element_type=jnp.float32)
        logits = logits + b_ref[:, pl.ds(c * CH, CH)]
        i = jax.lax.bitcast_convert_type(logits, jnp.int32)
        k = jnp.where(i < 0, i ^ MASK, i)
        keys_ref[:, pl.ds(c * CH, CH)] = k
        m1 = jnp.max(k, axis=1, keepdims=True)
        m2 = jnp.max(jnp.where(k == m1, SIGN, k), axis=1, keepdims=True)
        return jnp.maximum(hi, m1), jnp.minimum(lo, m2)

    hi, lo = jax.lax.fori_loop(
        0, NCH, phase_a,
        (jnp.full((BT, 1), SIGN, jnp.int32),
         jnp.full((BT, 1), MASK, jnp.int32)))

    def count_ge(cand):
        def count_chunk(c, cnt):
            k = keys_ref[:, pl.ds(c * CH2, CH2)]
            return cnt + jnp.sum((k >= cand).astype(jnp.int32), axis=1,
                                 keepdims=True)
        return jax.lax.fori_loop(0, NCH2, count_chunk,
                                 jnp.zeros((BT, 1), jnp.int32))

    def bs_cond(carry):
        hi, lo = carry
        return jnp.any(lo < hi)

    def bs_step(carry):
        hi, lo = carry
        half = jax.lax.shift_right_logical(hi - lo, 1)
        mid = ((hi ^ SIGN) - half) ^ SIGN
        cnt = count_ge(mid)
        ok = cnt >= K
        lo = jnp.where(ok, mid, lo)
        hi = jnp.where(ok, hi, ((mid ^ SIGN) - 1) ^ SIGN)
        return hi, lo

    hi, lo = jax.lax.while_loop(bs_cond, bs_step, (hi, lo))
    tk = lo

    def phase_c(c, acc):
        k = keys_ref[:, pl.ds(c * CH2, CH2)]
        i = jnp.where(k < 0, k ^ MASK, k)
        logit = jax.lax.bitcast_convert_type(i, jnp.float32)
        w = jnp.where(k >= tk, 1.0 / (1.0 + jnp.exp(-logit)), 0.0)
        tc = tab_ref[:, pl.ds(c * CH2, CH2)]
        return acc + jax.lax.dot_general(
            w, tc, (((1,), (1,)), ((), ())),
            preferred_element_type=jnp.float32)

    out_ref[...] = jax.lax.fori_loop(0, NCH2, phase_c,
                                     jnp.zeros((BT, E), jnp.float32))


def kernel(embed, W_enc, b_enc, table):
    B = embed.shape[0]
    pad = FPAD - NF
    wp = jnp.pad(W_enc.T, ((0, 0), (0, pad)))
    bp = jnp.pad(b_enc, (0, pad), constant_values=-1e30).reshape(1, FPAD)
    tp = jnp.pad(table.T, ((0, 0), (0, pad)))
    return pl.pallas_call(
        _body,
        grid=(B // BT,),
        in_specs=[
            pl.BlockSpec((BT, E), lambda i: (i, 0)),
            pl.BlockSpec((E, FPAD), lambda i: (0, 0)),
            pl.BlockSpec((1, FPAD), lambda i: (0, 0)),
            pl.BlockSpec((E, FPAD), lambda i: (0, 0)),
        ],
        out_specs=pl.BlockSpec((BT, E), lambda i: (i, 0)),
        out_shape=jax.ShapeDtypeStruct((B, E), jnp.float32),
        scratch_shapes=[pltpu.VMEM((BT, FPAD), jnp.int32)],
        compiler_params=pltpu.CompilerParams(
            dimension_semantics=("parallel",)),
    )(embed, wp, bp, tp)


# speedup vs baseline: 2.3510x; 1.1464x over previous
import jax
import jax.numpy as jnp
from jax.experimental import pallas as pl
from jax.experimental.pallas import tpu as pltpu

NF = 100000
E = 32
K = 64
BT = 32
CH = 2048
NCH = 49
FPAD = CH * NCH
CH2 = 14336
NCH2 = 7
MASK = 2**31 - 1
SIGN = -(2**31)


def _body(emb_ref, w_ref, b_ref, tab_ref, out_ref, keys_ref):
    emb = emb_ref[...]

    def phase_a(c, carry):
        hi, lo = carry
        wc = w_ref[:, pl.ds(c * CH, CH)]
        logits = jax.lax.dot_general(
            emb, wc, (((1,), (0,)), ((), ())),
            preferred_element_type=jnp.float32)
        logits = logits + b_ref[:, pl.ds(c * CH, CH)]
        i = jax.lax.bitcast_convert_type(logits, jnp.int32)
        k = jnp.where(i < 0, i ^ MASK, i)
        keys_ref[:, pl.ds(c * CH, CH)] = k
        m1 = jnp.max(k, axis=1, keepdims=True)
        m2 = jnp.max(jnp.where(k == m1, SIGN, k), axis=1, keepdims=True)
        return jnp.maximum(hi, m1), jnp.minimum(lo, m2)

    hi, lo = jax.lax.fori_loop(
        0, NCH, phase_a,
        (jnp.full((BT, 1), SIGN, jnp.int32),
         jnp.full((BT, 1), MASK, jnp.int32)))

    def count_ge(cand):
        def count_chunk(c, cnt):
            k = keys_ref[:, pl.ds(c * CH2, CH2)]
            return cnt + jnp.sum((k >= cand).astype(jnp.int32), axis=1,
                                 keepdims=True)
        return jax.lax.fori_loop(0, NCH2, count_chunk,
                                 jnp.zeros((BT, 1), jnp.int32))

    def bs_cond(carry):
        hi, lo = carry
        return jnp.any(lo < hi)

    def bs_step(carry):
        hi, lo = carry
        half = jax.lax.shift_right_logical(hi - lo, 1)
        mid = ((hi ^ SIGN) - half) ^ SIGN
        cnt = count_ge(mid)
        ok = cnt >= K
        lo = jnp.where(ok, mid, lo)
        hi = jnp.where(ok, hi, ((mid ^ SIGN) - 1) ^ SIGN)
        return hi, lo

    hi, lo = jax.lax.while_loop(bs_cond, bs_step, (hi, lo))
    tk = lo

    def phase_c(c, acc):
        k = keys_ref[:, pl.ds(c * CH2, CH2)]
        i = jnp.where(k < 0, k ^ MASK, k)
        logit = jax.lax.bitcast_convert_type(i, jnp.float32)
        w = jnp.where(k >= tk, 1.0 / (1.0 + jnp.exp(-logit)), 0.0)
        tc = tab_ref[:, pl.ds(c * CH2, CH2)]
        return acc + jax.lax.dot_general(
            w, tc, (((1,), (1,)), ((), ())),
            preferred_element_type=jnp.float32)

    out_ref[...] = jax.lax.fori_loop(0, NCH2, phase_c,
                                     jnp.zeros((BT, E), jnp.float32))


def kernel(embed, W_enc, b_enc, table):
    B = embed.shape[0]
    pad = FPAD - NF
    wp = jnp.pad(W_enc.T, ((0, 0), (0, pad)))
    bp = jnp.pad(b_enc, (0, pad), constant_values=-1e30).reshape(1, FPAD)
    tp = jnp.pad(table.T, ((0, 0), (0, pad)))
    return pl.pallas_call(
        _body,
        grid=(B // BT,),
        in_specs=[
            pl.BlockSpec((BT, E), lambda i: (i, 0)),
            pl.BlockSpec((E, FPAD), lambda i: (0, 0)),
            pl.BlockSpec((1, FPAD), lambda i: (0, 0)),
            pl.BlockSpec((E, FPAD), lambda i: (0, 0)),
        ],
        out_specs=pl.BlockSpec((BT, E), lambda i: (i, 0)),
        out_shape=jax.ShapeDtypeStruct((B, E), jnp.float32),
        scratch_shapes=[pltpu.VMEM((BT, FPAD), jnp.int32)],
        compiler_params=pltpu.CompilerParams(
            dimension_semantics=("parallel",)),
    )(embed, wp, bp, tp)


# speedup vs baseline: 2.6749x; 1.1378x over previous
import jax
import jax.numpy as jnp
from jax.experimental import pallas as pl
from jax.experimental.pallas import tpu as pltpu

NF = 100000
E = 32
K = 64
BT = 32
CH = 2048
NCH = 49
FPAD = CH * NCH
CH2 = 14336
NCH2 = 7
MASK = 2**31 - 1
SIGN = -(2**31)


def _body(emb_ref, w_ref, b_ref, tab_ref, out_ref, keys_ref):
    emb = emb_ref[...]

    def phase_a(c, carry):
        hi, lo = carry
        wc = w_ref[:, pl.ds(c * CH, CH)]
        logits = jax.lax.dot_general(
            emb, wc, (((1,), (0,)), ((), ())),
            preferred_element_type=jnp.float32)
        logits = logits + b_ref[:, pl.ds(c * CH, CH)]
        i = jax.lax.bitcast_convert_type(logits, jnp.int32)
        k = jnp.where(i < 0, i ^ MASK, i)
        keys_ref[:, pl.ds(c * CH, CH)] = k
        m1 = jnp.max(k, axis=1, keepdims=True)
        m2 = jnp.max(jnp.where(k == m1, SIGN, k), axis=1, keepdims=True)
        return jnp.maximum(hi, m1), jnp.minimum(lo, m2)

    hi, lo = jax.lax.fori_loop(
        0, NCH, phase_a,
        (jnp.full((BT, 1), SIGN, jnp.int32),
         jnp.full((BT, 1), MASK, jnp.int32)))

    def count_ge(cand):
        k = keys_ref[...]
        return jnp.sum((k >= cand).astype(jnp.int32), axis=1, keepdims=True)

    def bs_cond(carry):
        hi, lo = carry
        return jnp.any(lo < hi)

    def bs_step(carry):
        hi, lo = carry
        half = jax.lax.shift_right_logical(hi - lo, 1)
        mid = ((hi ^ SIGN) - half) ^ SIGN
        cnt = count_ge(mid)
        ok = cnt >= K
        lo = jnp.where(ok, mid, lo)
        hi = jnp.where(ok, hi, ((mid ^ SIGN) - 1) ^ SIGN)
        return hi, lo

    hi, lo = jax.lax.while_loop(bs_cond, bs_step, (hi, lo))
    tk = lo

    def phase_c(c, acc):
        k = keys_ref[:, pl.ds(c * CH2, CH2)]
        i = jnp.where(k < 0, k ^ MASK, k)
        logit = jax.lax.bitcast_convert_type(i, jnp.float32)
        w = jnp.where(k >= tk, 1.0 / (1.0 + jnp.exp(-logit)), 0.0)
        tc = tab_ref[:, pl.ds(c * CH2, CH2)]
        return acc + jax.lax.dot_general(
            w, tc, (((1,), (1,)), ((), ())),
            preferred_element_type=jnp.float32)

    out_ref[...] = jax.lax.fori_loop(0, NCH2, phase_c,
                                     jnp.zeros((BT, E), jnp.float32))


def kernel(embed, W_enc, b_enc, table):
    B = embed.shape[0]
    pad = FPAD - NF
    wp = jnp.pad(W_enc.T, ((0, 0), (0, pad)))
    bp = jnp.pad(b_enc, (0, pad), constant_values=-1e30).reshape(1, FPAD)
    tp = jnp.pad(table.T, ((0, 0), (0, pad)))
    return pl.pallas_call(
        _body,
        grid=(B // BT,),
        in_specs=[
            pl.BlockSpec((BT, E), lambda i: (i, 0)),
            pl.BlockSpec((E, FPAD), lambda i: (0, 0)),
            pl.BlockSpec((1, FPAD), lambda i: (0, 0)),
            pl.BlockSpec((E, FPAD), lambda i: (0, 0)),
        ],
        out_specs=pl.BlockSpec((BT, E), lambda i: (i, 0)),
        out_shape=jax.ShapeDtypeStruct((B, E), jnp.float32),
        scratch_shapes=[pltpu.VMEM((BT, FPAD), jnp.int32)],
        compiler_params=pltpu.CompilerParams(
            dimension_semantics=("parallel",)),
    )(embed, wp, bp, tp)


# speedup vs baseline: 2.6867x; 1.0044x over previous
import jax
import jax.numpy as jnp
from jax.experimental import pallas as pl
from jax.experimental.pallas import tpu as pltpu

NF = 100000
E = 32
K = 64
BT = 32
CH = 2048
NCH = 49
FPAD = CH * NCH
CH2 = 14336
NCH2 = 7
MASK = 2**31 - 1
SIGN = -(2**31)


def _body(emb_ref, w_ref, b_ref, tab_ref, out_ref, keys_ref):
    emb = emb_ref[...]

    def phase_a(c, carry):
        hi, lo = carry
        wc = w_ref[:, pl.ds(c * CH, CH)]
        logits = jax.lax.dot_general(
            emb, wc, (((1,), (0,)), ((), ())),
            preferred_element_type=jnp.float32)
        logits = logits + b_ref[:, pl.ds(c * CH, CH)]
        i = jax.lax.bitcast_convert_type(logits, jnp.int32)
        k = jnp.where(i < 0, i ^ MASK, i)
        keys_ref[:, pl.ds(c * CH, CH)] = k
        m1 = jnp.max(k, axis=1, keepdims=True)
        m2 = jnp.max(jnp.where(k == m1, SIGN, k), axis=1, keepdims=True)
        return jnp.maximum(hi, m1), jnp.minimum(lo, m2)

    hi, lo = jax.lax.fori_loop(
        0, NCH, phase_a,
        (jnp.full((BT, 1), SIGN, jnp.int32),
         jnp.full((BT, 1), MASK, jnp.int32)))

    def count_ge(cand):
        k = keys_ref[...]
        return jnp.sum((k >= cand).astype(jnp.int32), axis=1, keepdims=True)

    def bs_cond(carry):
        hi, lo = carry
        return jnp.any(lo < hi)

    def bs_step(carry):
        hi, lo = carry
        half = jax.lax.shift_right_logical(hi - lo, 1)
        mid = ((hi ^ SIGN) - half) ^ SIGN
        cnt = count_ge(mid)
        ok = cnt >= K
        lo = jnp.where(ok, mid, lo)
        hi = jnp.where(ok, hi, ((mid ^ SIGN) - 1) ^ SIGN)
        return hi, lo

    hi, lo = jax.lax.while_loop(bs_cond,
                                lambda c: bs_step(bs_step(c)), (hi, lo))
    tk = lo

    def phase_c(c, acc):
        k = keys_ref[:, pl.ds(c * CH2, CH2)]
        i = jnp.where(k < 0, k ^ MASK, k)
        logit = jax.lax.bitcast_convert_type(i, jnp.float32)
        w = jnp.where(k >= tk, 1.0 / (1.0 + jnp.exp(-logit)), 0.0)
        tc = tab_ref[:, pl.ds(c * CH2, CH2)]
        return acc + jax.lax.dot_general(
            w, tc, (((1,), (1,)), ((), ())),
            preferred_element_type=jnp.float32)

    out_ref[...] = jax.lax.fori_loop(0, NCH2, phase_c,
                                     jnp.zeros((BT, E), jnp.float32))


def kernel(embed, W_enc, b_enc, table):
    B = embed.shape[0]
    pad = FPAD - NF
    wp = jnp.pad(W_enc.T, ((0, 0), (0, pad)))
    bp = jnp.pad(b_enc, (0, pad), constant_values=-1e30).reshape(1, FPAD)
    tp = jnp.pad(table.T, ((0, 0), (0, pad)))
    return pl.pallas_call(
        _body,
        grid=(B // BT,),
        in_specs=[
            pl.BlockSpec((BT, E), lambda i: (i, 0)),
            pl.BlockSpec((E, FPAD), lambda i: (0, 0)),
            pl.BlockSpec((1, FPAD), lambda i: (0, 0)),
            pl.BlockSpec((E, FPAD), lambda i: (0, 0)),
        ],
        out_specs=pl.BlockSpec((BT, E), lambda i: (i, 0)),
        out_shape=jax.ShapeDtypeStruct((B, E), jnp.float32),
        scratch_shapes=[pltpu.VMEM((BT, FPAD), jnp.int32)],
        compiler_params=pltpu.CompilerParams(
            dimension_semantics=("parallel",)),
    )(embed, wp, bp, tp)
